# baseline (speedup 1.0000x reference)
import jax
import jax.numpy as jnp
from jax import lax
from jax.experimental import pallas as pl
from jax.experimental.pallas import tpu as pltpu
from jax.experimental.pallas import tpu_sc as plsc

K_ACTIVE = 512
ROWS = 64
N = 8192
LANES = 16
NCHUNK = N // LANES
NBINS = 256
HISTW = NBINS * LANES
TSTRIDE = 17
NSTATIC = 2
INT_MIN = -2147483648
LOW31 = 2147483647


def _encode(xv):
    b = lax.bitcast_convert_type(xv, jnp.int32)
    e = b ^ (lax.shift_right_arithmetic(b, 31) & LOW31)
    return e ^ INT_MIN


def _decode(eu):
    e = eu ^ INT_MIN
    b = e ^ (lax.shift_right_arithmetic(e, 31) & LOW31)
    return lax.bitcast_convert_type(b, jnp.float32)


def _suffix_sum(v):
    r = lax.rev(v, (0,))
    return lax.rev(plsc.cumsum(r), (0,))


def _bin1(xv):
    y = lax.convert_element_type(xv * 8.0 + 1152.0, jnp.int32) - 1024
    return jnp.clip(y, 0, NBINS - 1)


def _bin2(xv, lov):
    y = lax.convert_element_type((xv - lov) * 2048.0, jnp.int32)
    return jnp.clip(y, 0, NBINS - 1)


def _kwta_sc_body(x_hbm, h_hbm, o_hbm, xrow, comb, hist, totbuf, orow, hbuf):
    wid = lax.axis_index("s") * 2 + lax.axis_index("c")
    pltpu.sync_copy(h_hbm, hbuf)
    lane = lax.iota(jnp.int32, LANES)
    ones = jnp.ones((LANES,), jnp.int32)

    def splat(s, dtype=jnp.int32):
        return jnp.broadcast_to(jnp.asarray(s, dtype), (LANES,))

    def rank_locate(kv, binofs):
        tch = jnp.zeros((LANES,), jnp.int32)
        for l in range(LANES):
            tch = tch + plsc.load_gather(
                totbuf, [(binofs + lane * LANES + l) * TSTRIDE + lane])
        st = _suffix_sum(tch)
        cs = plsc.all_reduce_population_count(st > kv) - 1
        agt = splat(jnp.sum(jnp.where(lane == cs, st - tch, 0)))
        tot_c = plsc.load_gather(
            totbuf, [(binofs + cs * LANES + lane) * TSTRIDE])
        s2 = _suffix_sum(tot_c)
        inb = plsc.all_reduce_population_count((agt + s2) > kv) - 1
        dv = cs * LANES + inb
        above = agt + splat(jnp.sum(jnp.where(lane == inb, s2 - tot_c, 0)))
        return dv, above

    for r in range(2):
        row = wid * 2 + r
        base = row * N
        pltpu.sync_copy(x_hbm.at[pl.ds(base, N)], xrow)

        def zbody(i):
            hist[pl.ds(i * LANES, LANES)] = jnp.zeros((LANES,), jnp.int32)

        plsc.parallel_loop(0, 3 * NBINS, unroll=8)(zbody)

        def p1body(i):
            xv = xrow[pl.ds(i * LANES, LANES)]
            idx = _bin1(xv) * LANES + lane
            plsc.addupdate_scatter(hist, [idx], ones)

        plsc.parallel_loop(0, NCHUNK, unroll=8)(p1body)

        def t1body(i):
            h = hist[pl.ds(i * LANES, LANES)]
            totbuf[pl.ds(i * TSTRIDE, LANES)] = splat(jnp.sum(h))

        plsc.parallel_loop(0, NBINS, unroll=8)(t1body)

        d1v, a1v = rank_locate(splat(K_ACTIVE), 0)
        d2v, a2v = rank_locate(splat(K_ACTIVE + 1), 0)
        samebin = d1v == d2v
        lo1 = lax.convert_element_type(d1v, jnp.float32) * 0.125 - 16.0
        lo2 = lax.convert_element_type(d2v, jnp.float32) * 0.125 - 16.0

        def p2body(i):
            xv = xrow[pl.ds(i * LANES, LANES)]
            b1 = _bin1(xv)
            mA = b1 == d1v
            mB = b1 == d2v
            m = jnp.logical_or(mA, mB)
            b2 = jnp.where(mA, _bin2(xv, lo1), _bin2(xv, lo2))
            sel = jnp.where(mA, HISTW, 2 * HISTW)
            idx = sel + b2 * LANES + lane
            plsc.addupdate_scatter(hist, [idx], ones, mask=m)

        plsc.parallel_loop(0, NCHUNK, unroll=8)(p2body)

        def t2body(i):
            h = hist[pl.ds(HISTW + i * LANES, LANES)]
            totbuf[pl.ds((NBINS + i) * TSTRIDE, LANES)] = splat(jnp.sum(h))

        plsc.parallel_loop(0, 2 * NBINS, unroll=8)(t2body)

        e1v, ab1v = rank_locate(splat(K_ACTIVE) - a1v, splat(NBINS))
        regB = jnp.where(samebin, splat(NBINS), splat(2 * NBINS))
        kB = jnp.where(samebin, splat(K_ACTIVE + 1) - a1v,
                       splat(K_ACTIVE + 1) - a2v)
        e2v, _ = rank_locate(kB, regB)
        above = a1v + ab1v

        def p3body(i, off):
            xv = xrow[pl.ds(i * LANES, LANES)]
            b1 = _bin1(xv)
            mA = jnp.logical_and(b1 == d1v, _bin2(xv, lo1) == e1v)
            mB = jnp.logical_and(b1 == d2v, _bin2(xv, lo2) == e2v)
            m = jnp.logical_or(mA, mB)
            mi = m.astype(jnp.int32)
            idx = off + plsc.cumsum(mi) - mi
            plsc.store_scatter(comb, [idx], _encode(xv), mask=m)
            return off + plsc.all_reduce_population_count(m)

        offv = plsc.parallel_loop(
            0, NCHUNK, unroll=4,
            carry=jnp.zeros((LANES,), jnp.int32))(p3body)
        cnt = offv[0]
        nch = (cnt + LANES - 1) // LANES

        def make_bsbody(with_tail):
            def bsbody(t, carry):
                p1v, k1v, p2v, k2v = carry
                iv = splat(31) - splat(t)
                bitv = lax.shift_left(ones, iv)
                mhv = lax.shift_left(splat(-1), iv)
                t1v = p1v | bitv
                t2v = p2v | bitv

                def cbody(jj, cc):
                    c1, c2 = cc
                    v = comb[pl.ds(jj * LANES, LANES)]
                    valid = (jj * LANES + lane) < offv
                    vm = v & mhv
                    m1 = jnp.logical_and(vm == t1v, valid)
                    m2 = jnp.logical_and(vm == t2v, valid)
                    return (c1 + plsc.all_reduce_population_count(m1),
                            c2 + plsc.all_reduce_population_count(m2))

                zv = jnp.zeros((LANES,), jnp.int32)
                cc = (zv, zv)
                for jj in range(NSTATIC):
                    cc = cbody(jj, cc)
                if with_tail:
                    cc = lax.fori_loop(NSTATIC, nch, cbody, cc)
                c1, c2 = cc
                take1 = k1v < c1
                p1v = jnp.where(take1, t1v, p1v)
                k1v = jnp.where(take1, k1v, k1v - c1)
                take2 = k2v < c2
                p2v = jnp.where(take2, t2v, p2v)
                k2v = jnp.where(take2, k2v, k2v - c2)
                return p1v, k1v, p2v, k2v
            return bsbody

        zv = jnp.zeros((LANES,), jnp.int32)
        bs_init = (zv, splat(K_ACTIVE) - above,
                   zv, splat(K_ACTIVE + 1) - above)
        p1v, _, p2v, _ = lax.cond(
            cnt <= NSTATIC * LANES,
            lambda: lax.fori_loop(0, 32, make_bsbody(False), bs_init),
            lambda: lax.fori_loop(0, 32, make_bsbody(True), bs_init))

        thr = (_decode(p1v) + _decode(p2v)) * 0.5
        hv = hbuf[...]

        def sgbody(i):
            xv = xrow[pl.ds(i * LANES, LANES)]
            zz = hv * (xv - thr)
            orow[pl.ds(i * LANES, LANES)] = 1.0 / (1.0 + jnp.exp(-zz))

        plsc.parallel_loop(0, NCHUNK, unroll=8)(sgbody)
        pltpu.sync_copy(orow, o_hbm.at[pl.ds(base, N)])


@jax.jit
def _kwta_sc(x_flat, h_vec):
    mesh = plsc.VectorSubcoreMesh(
        core_axis_name="c", subcore_axis_name="s", num_cores=2,
        num_subcores=16)
    f = pl.kernel(
        _kwta_sc_body,
        out_type=jax.ShapeDtypeStruct((ROWS * N,), jnp.float32),
        mesh=mesh,
        scratch_types=[
            pltpu.VMEM((N,), jnp.float32),
            pltpu.VMEM((N + LANES,), jnp.int32),
            pltpu.VMEM((3 * HISTW,), jnp.int32),
            pltpu.VMEM((3 * NBINS * TSTRIDE + LANES,), jnp.int32),
            pltpu.VMEM((N,), jnp.float32),
            pltpu.VMEM((LANES,), jnp.float32),
        ],
        compiler_params=pltpu.CompilerParams(
            needs_layout_passes=False, skip_device_barrier=True,
            disable_bounds_checks=True, disable_semaphore_checks=True),
    )
    return f(x_flat, h_vec)


def kernel(x, hardness):
    x_flat = jnp.reshape(x, (ROWS * N,))
    h_vec = jnp.full((LANES,), hardness, jnp.float32)
    out = _kwta_sc(x_flat, h_vec)
    return jnp.reshape(out, (ROWS, N))


# speedup vs baseline: 1.0122x; 1.0122x over previous
import jax
import jax.numpy as jnp
from jax import lax
from jax.experimental import pallas as pl
from jax.experimental.pallas import tpu as pltpu
from jax.experimental.pallas import tpu_sc as plsc

K_ACTIVE = 512
ROWS = 64
N = 8192
LANES = 16
NCHUNK = N // LANES
NBINS = 256
HISTW = NBINS * LANES
TSTRIDE = 17
NSTATIC = 2
INT_MIN = -2147483648
LOW31 = 2147483647


def _encode(xv):
    b = lax.bitcast_convert_type(xv, jnp.int32)
    e = b ^ (lax.shift_right_arithmetic(b, 31) & LOW31)
    return e ^ INT_MIN


def _decode(eu):
    e = eu ^ INT_MIN
    b = e ^ (lax.shift_right_arithmetic(e, 31) & LOW31)
    return lax.bitcast_convert_type(b, jnp.float32)


def _suffix_sum(v):
    r = lax.rev(v, (0,))
    return lax.rev(plsc.cumsum(r), (0,))


def _bin1(xv):
    y = lax.convert_element_type(xv * 8.0 + 1152.0, jnp.int32) - 1024
    return jnp.clip(y, 0, NBINS - 1)


def _bin2(xv, lov):
    y = lax.convert_element_type((xv - lov) * 2048.0, jnp.int32)
    return jnp.clip(y, 0, NBINS - 1)


def _kwta_sc_body(x_hbm, h_hbm, o_hbm, xrow, comb, hist, totbuf, orow, hbuf):
    wid = lax.axis_index("s") * 2 + lax.axis_index("c")
    pltpu.sync_copy(h_hbm, hbuf)
    lane = lax.iota(jnp.int32, LANES)
    ones = jnp.ones((LANES,), jnp.int32)

    def splat(s, dtype=jnp.int32):
        return jnp.broadcast_to(jnp.asarray(s, dtype), (LANES,))

    def rank_locate(kv, binofs):
        tch = jnp.zeros((LANES,), jnp.int32)
        for l in range(LANES):
            tch = tch + plsc.load_gather(
                totbuf, [(binofs + lane * LANES + l) * TSTRIDE + lane])
        st = _suffix_sum(tch)
        cs = plsc.all_reduce_population_count(st > kv) - 1
        agt = splat(jnp.sum(jnp.where(lane == cs, st - tch, 0)))
        tot_c = plsc.load_gather(
            totbuf, [(binofs + cs * LANES + lane) * TSTRIDE])
        s2 = _suffix_sum(tot_c)
        inb = plsc.all_reduce_population_count((agt + s2) > kv) - 1
        dv = cs * LANES + inb
        above = agt + splat(jnp.sum(jnp.where(lane == inb, s2 - tot_c, 0)))
        return dv, above

    def rank_locate2(kva, kvb, binofs):
        tch = jnp.zeros((LANES,), jnp.int32)
        for l in range(LANES):
            tch = tch + plsc.load_gather(
                totbuf, [(binofs + lane * LANES + l) * TSTRIDE + lane])
        st = _suffix_sum(tch)

        def one(kv):
            cs = plsc.all_reduce_population_count(st > kv) - 1
            agt = splat(jnp.sum(jnp.where(lane == cs, st - tch, 0)))
            tot_c = plsc.load_gather(
                totbuf, [(binofs + cs * LANES + lane) * TSTRIDE])
            s2 = _suffix_sum(tot_c)
            inb = plsc.all_reduce_population_count((agt + s2) > kv) - 1
            dv = cs * LANES + inb
            above = agt + splat(
                jnp.sum(jnp.where(lane == inb, s2 - tot_c, 0)))
            return dv, above

        dva, aa = one(kva)
        dvb, ab = one(kvb)
        return dva, aa, dvb, ab

    pltpu.sync_copy(x_hbm.at[pl.ds(wid * 2 * N, 2 * N)], xrow)
    for r in range(2):
        rb = r * N

        def zbody(i):
            hist[pl.ds(i * LANES, LANES)] = jnp.zeros((LANES,), jnp.int32)

        plsc.parallel_loop(0, 3 * NBINS, unroll=8)(zbody)

        def p1body(i):
            xv = xrow[pl.ds(rb + i * LANES, LANES)]
            idx = _bin1(xv) * LANES + lane
            plsc.addupdate_scatter(hist, [idx], ones)

        plsc.parallel_loop(0, NCHUNK, unroll=8)(p1body)

        def t1body(i):
            h = hist[pl.ds(i * LANES, LANES)]
            totbuf[pl.ds(i * TSTRIDE, LANES)] = splat(jnp.sum(h))

        plsc.parallel_loop(0, NBINS, unroll=8)(t1body)

        d1v, a1v, d2v, a2v = rank_locate2(
            splat(K_ACTIVE), splat(K_ACTIVE + 1), 0)
        samebin = d1v == d2v
        lo1 = lax.convert_element_type(d1v, jnp.float32) * 0.125 - 16.0
        lo2 = lax.convert_element_type(d2v, jnp.float32) * 0.125 - 16.0

        def p2body(i):
            xv = xrow[pl.ds(rb + i * LANES, LANES)]
            b1 = _bin1(xv)
            mA = b1 == d1v
            mB = b1 == d2v
            m = jnp.logical_or(mA, mB)
            b2 = jnp.where(mA, _bin2(xv, lo1), _bin2(xv, lo2))
            sel = jnp.where(mA, HISTW, 2 * HISTW)
            idx = sel + b2 * LANES + lane
            plsc.addupdate_scatter(hist, [idx], ones, mask=m)

        plsc.parallel_loop(0, NCHUNK, unroll=8)(p2body)

        def t2body(i):
            h = hist[pl.ds(HISTW + i * LANES, LANES)]
            totbuf[pl.ds((NBINS + i) * TSTRIDE, LANES)] = splat(jnp.sum(h))

        plsc.parallel_loop(0, 2 * NBINS, unroll=8)(t2body)

        e1v, ab1v = rank_locate(splat(K_ACTIVE) - a1v, splat(NBINS))
        regB = jnp.where(samebin, splat(NBINS), splat(2 * NBINS))
        kB = jnp.where(samebin, splat(K_ACTIVE + 1) - a1v,
                       splat(K_ACTIVE + 1) - a2v)
        e2v, _ = rank_locate(kB, regB)
        above = a1v + ab1v

        def p3body(i, off):
            xv = xrow[pl.ds(rb + i * LANES, LANES)]
            b1 = _bin1(xv)
            mA = jnp.logical_and(b1 == d1v, _bin2(xv, lo1) == e1v)
            mB = jnp.logical_and(b1 == d2v, _bin2(xv, lo2) == e2v)
            m = jnp.logical_or(mA, mB)
            mi = m.astype(jnp.int32)
            idx = off + plsc.cumsum(mi) - mi
            plsc.store_scatter(comb, [idx], _encode(xv), mask=m)
            return off + plsc.all_reduce_population_count(m)

        offv = plsc.parallel_loop(
            0, NCHUNK, unroll=4,
            carry=jnp.zeros((LANES,), jnp.int32))(p3body)
        cnt = offv[0]
        nch = (cnt + LANES - 1) // LANES

        def make_bsbody(with_tail):
            def bsbody(t, carry):
                p1v, k1v, p2v, k2v = carry
                iv = splat(31) - splat(t)
                bitv = lax.shift_left(ones, iv)
                mhv = lax.shift_left(splat(-1), iv)
                t1v = p1v | bitv
                t2v = p2v | bitv

                def cbody(jj, cc):
                    c1, c2 = cc
                    v = comb[pl.ds(jj * LANES, LANES)]
                    valid = (jj * LANES + lane) < offv
                    vm = v & mhv
                    m1 = jnp.logical_and(vm == t1v, valid)
                    m2 = jnp.logical_and(vm == t2v, valid)
                    return (c1 + plsc.all_reduce_population_count(m1),
                            c2 + plsc.all_reduce_population_count(m2))

                zv = jnp.zeros((LANES,), jnp.int32)
                cc = (zv, zv)
                for jj in range(NSTATIC):
                    cc = cbody(jj, cc)
                if with_tail:
                    cc = lax.fori_loop(NSTATIC, nch, cbody, cc)
                c1, c2 = cc
                take1 = k1v < c1
                p1v = jnp.where(take1, t1v, p1v)
                k1v = jnp.where(take1, k1v, k1v - c1)
                take2 = k2v < c2
                p2v = jnp.where(take2, t2v, p2v)
                k2v = jnp.where(take2, k2v, k2v - c2)
                return p1v, k1v, p2v, k2v
            return bsbody

        zv = jnp.zeros((LANES,), jnp.int32)
        bs_init = (zv, splat(K_ACTIVE) - above,
                   zv, splat(K_ACTIVE + 1) - above)
        p1v, _, p2v, _ = lax.cond(
            cnt <= NSTATIC * LANES,
            lambda: lax.fori_loop(0, 32, make_bsbody(False), bs_init),
            lambda: lax.fori_loop(0, 32, make_bsbody(True), bs_init))

        thr = (_decode(p1v) + _decode(p2v)) * 0.5
        hv = hbuf[...]

        def sgbody(i):
            xv = xrow[pl.ds(rb + i * LANES, LANES)]
            zz = hv * (xv - thr)
            orow[pl.ds(rb + i * LANES, LANES)] = 1.0 / (1.0 + jnp.exp(-zz))

        plsc.parallel_loop(0, NCHUNK, unroll=8)(sgbody)
    pltpu.sync_copy(orow, o_hbm.at[pl.ds(wid * 2 * N, 2 * N)])


@jax.jit
def _kwta_sc(x_flat, h_vec):
    mesh = plsc.VectorSubcoreMesh(
        core_axis_name="c", subcore_axis_name="s", num_cores=2,
        num_subcores=16)
    f = pl.kernel(
        _kwta_sc_body,
        out_type=jax.ShapeDtypeStruct((ROWS * N,), jnp.float32),
        mesh=mesh,
        scratch_types=[
            pltpu.VMEM((2 * N,), jnp.float32),
            pltpu.VMEM((N + LANES,), jnp.int32),
            pltpu.VMEM((3 * HISTW,), jnp.int32),
            pltpu.VMEM((3 * NBINS * TSTRIDE + LANES,), jnp.int32),
            pltpu.VMEM((2 * N,), jnp.float32),
            pltpu.VMEM((LANES,), jnp.float32),
        ],
        compiler_params=pltpu.CompilerParams(needs_layout_passes=False),
    )
    return f(x_flat, h_vec)


def kernel(x, hardness):
    x_flat = jnp.reshape(x, (ROWS * N,))
    h_vec = jnp.full((LANES,), hardness, jnp.float32)
    out = _kwta_sc(x_flat, h_vec)
    return jnp.reshape(out, (ROWS, N))
